# baseline (speedup 1.0000x reference)
import functools

import jax
import jax.numpy as jnp
from jax import lax
from jax.experimental import pallas as pl
from jax.experimental.pallas import tpu as pltpu
from jax.experimental.pallas import tpu_sc as plsc

_ALPHA = 0.25
_N = 2048
_V = 32000
_VPAD = 32768
_RB = 64
_CB = 32000
_LANES = 16
_SC_CORES = 1


def _fo_body(tgt_hbm, fo_hbm, tgt_v, fo_v):
    wid = lax.axis_index("s") * _SC_CORES + lax.axis_index("c")
    per_w = _VPAD // (_SC_CORES * 16)
    base = wid * per_w

    pltpu.sync_copy(tgt_hbm, tgt_v)

    def init(k, carry):
        fo_v[pl.ds(k * _LANES, _LANES)] = jnp.full((_LANES,), _N, jnp.int32)
        return carry

    lax.fori_loop(0, per_w // _LANES, init, 0)

    lanes = jnp.arange(_LANES, dtype=jnp.int32)
    prev_idx = jnp.maximum(lanes - 1, 0)

    unroll = 4

    def body(k, carry):
        for u in range(unroll):
            jb = (_N - _LANES) - (k * unroll + u) * _LANES
            v = tgt_v[pl.ds(jb, _LANES)]
            j = jb + lanes
            key = (v << 11) | j
            skey, js = plsc.sort_key_val(key, j)
            vs = skey >> 11
            pv = lax.gather(
                vs,
                prev_idx[:, None],
                lax.GatherDimensionNumbers(
                    offset_dims=(),
                    collapsed_slice_dims=(0,),
                    start_index_map=(0,),
                ),
                slice_sizes=(1,),
                mode=lax.GatherScatterMode.PROMISE_IN_BOUNDS,
            )
            first = (vs != pv) | (lanes == 0)
            rm = (vs >= base) & (vs < base + per_w)
            plsc.store_scatter(fo_v, [vs - base], js, mask=first & rm)
        return carry

    lax.fori_loop(0, _N // (_LANES * unroll), body, 0)
    pltpu.sync_copy(fo_v, fo_hbm.at[pl.ds(base, per_w)])


def _make_fo_kernel():
    mesh = plsc.VectorSubcoreMesh(
        core_axis_name="c", subcore_axis_name="s", num_cores=_SC_CORES
    )
    per_w = _VPAD // (_SC_CORES * 16)
    return functools.partial(
        pl.kernel,
        mesh=mesh,
        out_type=jax.ShapeDtypeStruct((_VPAD,), jnp.int32),
        scratch_types=[
            pltpu.VMEM((_N,), jnp.int32),
            pltpu.VMEM((per_w,), jnp.int32),
        ],
        compiler_params=pltpu.CompilerParams(needs_layout_passes=False),
    )(_fo_body)


def _loss_body(t_ref, fo_ref, lp_ref, out_ref):
    ri = pl.program_id(0)
    ci = pl.program_id(1)

    @pl.when((ri == 0) & (ci == 0))
    def _():
        out_ref[0, 0] = 0.0

    lp = lp_ref[...]
    t = t_ref[...]
    fo = fo_ref[...]
    i_ids = ri * _RB + lax.broadcasted_iota(jnp.int32, (_RB, _CB), 0)
    v_ids = ci * _CB + lax.broadcasted_iota(jnp.int32, (_RB, _CB), 1)
    mask = ((fo < i_ids) & (v_ids != t)) | (v_ids == 0)
    f = -jnp.log(jnp.maximum(1.0 - jnp.exp(lp), 1e-5))
    out_ref[0, 0] += _ALPHA * jnp.sum(jnp.where(mask, f, 0.0))


_loss_call = pl.pallas_call(
    _loss_body,
    grid=(_N // _RB, _V // _CB),
    in_specs=[
        pl.BlockSpec((_RB, 1), lambda ri, ci: (ri, 0)),
        pl.BlockSpec((1, _CB), lambda ri, ci: (0, ci)),
        pl.BlockSpec((_RB, _CB), lambda ri, ci: (ri, ci)),
    ],
    out_specs=pl.BlockSpec(
        (1, 1), lambda ri, ci: (0, 0), memory_space=pltpu.SMEM
    ),
    out_shape=jax.ShapeDtypeStruct((1, 1), jnp.float32),
)


def kernel(logprobs, target):
    target = target.reshape(-1).astype(jnp.int32)
    lp = logprobs.reshape(-1, logprobs.shape[-1])
    fo = _make_fo_kernel()(target)
    fo2 = fo.reshape(1, _VPAD)
    t2 = target.reshape(_N, 1)
    loss = _loss_call(t2, fo2, lp)
    return loss[0, 0]


# speedup vs baseline: 1.0090x; 1.0090x over previous
import functools

import jax
import jax.numpy as jnp
from jax import lax
from jax.experimental import pallas as pl
from jax.experimental.pallas import tpu as pltpu
from jax.experimental.pallas import tpu_sc as plsc

_ALPHA = 0.25
_N = 2048
_V = 32000
_VPAD = 32768
_RB = 128
_CB = 32000
_LANES = 16
_SC_CORES = 1


def _fo_body(tgt_hbm, fo_hbm, tgt_v, fo_v):
    wid = lax.axis_index("s") * _SC_CORES + lax.axis_index("c")
    per_w = _VPAD // (_SC_CORES * 16)
    base = wid * per_w

    pltpu.sync_copy(tgt_hbm, tgt_v)

    def init(k, carry):
        fo_v[pl.ds(k * _LANES, _LANES)] = jnp.full((_LANES,), _N, jnp.int32)
        return carry

    lax.fori_loop(0, per_w // _LANES, init, 0)

    lanes = jnp.arange(_LANES, dtype=jnp.int32)
    prev_idx = jnp.maximum(lanes - 1, 0)

    unroll = 4

    def body(k, carry):
        for u in range(unroll):
            jb = (_N - _LANES) - (k * unroll + u) * _LANES
            v = tgt_v[pl.ds(jb, _LANES)]
            j = jb + lanes
            key = (v << 11) | j
            skey, js = plsc.sort_key_val(key, j)
            vs = skey >> 11
            pv = lax.gather(
                vs,
                prev_idx[:, None],
                lax.GatherDimensionNumbers(
                    offset_dims=(),
                    collapsed_slice_dims=(0,),
                    start_index_map=(0,),
                ),
                slice_sizes=(1,),
                mode=lax.GatherScatterMode.PROMISE_IN_BOUNDS,
            )
            first = (vs != pv) | (lanes == 0)
            rm = (vs >= base) & (vs < base + per_w)
            plsc.store_scatter(fo_v, [vs - base], js, mask=first & rm)
        return carry

    lax.fori_loop(0, _N // (_LANES * unroll), body, 0)
    pltpu.sync_copy(fo_v, fo_hbm.at[pl.ds(base, per_w)])


def _make_fo_kernel():
    mesh = plsc.VectorSubcoreMesh(
        core_axis_name="c", subcore_axis_name="s", num_cores=_SC_CORES
    )
    per_w = _VPAD // (_SC_CORES * 16)
    return functools.partial(
        pl.kernel,
        mesh=mesh,
        out_type=jax.ShapeDtypeStruct((_VPAD,), jnp.int32),
        scratch_types=[
            pltpu.VMEM((_N,), jnp.int32),
            pltpu.VMEM((per_w,), jnp.int32),
        ],
        compiler_params=pltpu.CompilerParams(needs_layout_passes=False),
    )(_fo_body)


def _loss_body(t_ref, fo_ref, lp_ref, out_ref):
    ri = pl.program_id(0)
    ci = pl.program_id(1)

    @pl.when((ri == 0) & (ci == 0))
    def _():
        out_ref[0, 0] = 0.0

    lp = lp_ref[...]
    t = t_ref[...]
    fo = fo_ref[...]
    i_ids = ri * _RB + lax.broadcasted_iota(jnp.int32, (_RB, _CB), 0)
    v_ids = ci * _CB + lax.broadcasted_iota(jnp.int32, (_RB, _CB), 1)
    mask = ((fo < i_ids) & (v_ids != t)) | (v_ids == 0)
    f = -jnp.log(jnp.maximum(1.0 - jnp.exp(lp), 1e-5))
    out_ref[0, 0] += _ALPHA * jnp.sum(jnp.where(mask, f, 0.0))


_loss_call = pl.pallas_call(
    _loss_body,
    grid=(_N // _RB, _V // _CB),
    in_specs=[
        pl.BlockSpec((_RB, 1), lambda ri, ci: (ri, 0)),
        pl.BlockSpec((1, _CB), lambda ri, ci: (0, ci)),
        pl.BlockSpec((_RB, _CB), lambda ri, ci: (ri, ci)),
    ],
    out_specs=pl.BlockSpec(
        (1, 1), lambda ri, ci: (0, 0), memory_space=pltpu.SMEM
    ),
    out_shape=jax.ShapeDtypeStruct((1, 1), jnp.float32),
)


def kernel(logprobs, target):
    target = target.reshape(-1).astype(jnp.int32)
    lp = logprobs.reshape(-1, logprobs.shape[-1])
    fo = _make_fo_kernel()(target)
    fo2 = fo.reshape(1, _VPAD)
    t2 = target.reshape(_N, 1)
    loss = _loss_call(t2, fo2, lp)
    return loss[0, 0]
